# baseline (speedup 1.0000x reference)
import functools

import jax
import jax.numpy as jnp
from jax import lax
from jax.experimental import pallas as pl
from jax.experimental.pallas import tpu as pltpu
from jax.experimental.pallas import tpu_sc as plsc

N_NODES = 10000
N_EDGES = 320000
D = 128

NC = 2
NS = 16
NW = NC * NS
E_PER_TILE = 10240
E_PAD = NW * E_PER_TILE
CHUNK = 128
N_CHUNKS = E_PER_TILE // CHUNK
ACC_ROWS = 10240
ROWS_PER_TILE = ACC_ROWS // NS

NB = 1024
GRID = (N_NODES + NB - 1) // NB

_mesh = lambda: plsc.VectorSubcoreMesh(core_axis_name="c", subcore_axis_name="s")


NBUF = 2
CNBUF = 4
CN_GROUPS = N_CHUNKS // CNBUF

A_CH = 40
B_CH = 2 * N_CHUNKS - A_CH


@functools.partial(
    pl.kernel,
    out_type=jax.ShapeDtypeStruct((NW, ROWS_PER_TILE, D), jnp.float32),
    mesh=_mesh(),
    scratch_types=[
        [pltpu.VMEM((CHUNK,), jnp.int32)] * NBUF,
        [pltpu.VMEM((CHUNK,), jnp.int32)] * NBUF,
        [pltpu.VMEM((CHUNK, D), jnp.float32)] * NBUF,
        pltpu.VMEM_SHARED((ACC_ROWS, D), jnp.float32),
        [pltpu.SemaphoreType.DMA] * NBUF,
        [pltpu.SemaphoreType.DMA] * NBUF,
        [pltpu.SemaphoreType.DMA] * NBUF,
    ])
def _seg_sum(table, src, dst, zrows, psum_out, src_v, dst_v, rows_v, acc,
             sem_i, sem_g, sem_s):
  c = lax.axis_index("c")
  s = lax.axis_index("s")
  wid = c * NS + s
  arow = pl.multiple_of(s * ROWS_PER_TILE, ROWS_PER_TILE)
  pltpu.sync_copy(zrows, acc.at[pl.ds(arow, ROWS_PER_TILE)])
  plsc.subcore_barrier()

  ebase = jnp.where(c == 0, s * (A_CH * CHUNK),
                    NS * (A_CH * CHUNK) + s * (B_CH * CHUNK))
  n_groups = jnp.where(c == 0, A_CH // NBUF, B_CH // NBUF)

  def idx_start(k, b):
    off = pl.multiple_of(ebase + k * CHUNK, CHUNK)
    pltpu.async_copy(src.at[pl.ds(off, CHUNK)], src_v[b], sem_i[b])
    pltpu.async_copy(dst.at[pl.ds(off, CHUNK)], dst_v[b], sem_i[b])

  def idx_wait(b):
    pltpu.make_async_copy(src.at[pl.ds(0, CHUNK)], src_v[b], sem_i[b]).wait()
    pltpu.make_async_copy(dst.at[pl.ds(0, CHUNK)], dst_v[b], sem_i[b]).wait()

  def gather_start(b):
    pltpu.async_copy(table.at[src_v[b]], rows_v[b], sem_g[b])

  def gather_wait(b):
    pltpu.make_async_copy(table.at[src_v[b]], rows_v[b], sem_g[b]).wait()

  def scatter_start(b):
    pltpu.async_copy(rows_v[b], acc.at[dst_v[b]], sem_s[b], add=True)

  def scatter_wait(b):
    pltpu.make_async_copy(rows_v[b], acc.at[dst_v[b]], sem_s[b]).wait()

  for b in range(NBUF):
    idx_start(b, b)
  for b in range(NBUF):
    idx_wait(b)
    gather_start(b)

  def group(g, carry):
    k0 = g * NBUF
    for b in range(NBUF):
      gather_wait(b)
      scatter_start(b)
    for b in range(NBUF):
      scatter_wait(b)
      idx_start(k0 + NBUF + b, b)
    for b in range(NBUF):
      idx_wait(b)
      gather_start(b)
    return carry

  lax.fori_loop(0, n_groups - 1, group, 0)
  for b in range(NBUF):
    gather_wait(b)
    scatter_start(b)
  for b in range(NBUF):
    scatter_wait(b)

  plsc.subcore_barrier()
  pltpu.sync_copy(acc.at[pl.ds(arow, ROWS_PER_TILE)], psum_out.at[wid])


@functools.partial(
    pl.kernel,
    out_type=jax.ShapeDtypeStruct((NW, ROWS_PER_TILE, D), jnp.float32),
    mesh=_mesh(),
    scratch_types=[
        [pltpu.VMEM((CHUNK,), jnp.int32)] * CNBUF,
        pltpu.VMEM((CHUNK, D), jnp.float32),
        pltpu.VMEM_SHARED((ACC_ROWS, D), jnp.float32),
        [pltpu.SemaphoreType.DMA] * CNBUF,
        [pltpu.SemaphoreType.DMA] * CNBUF,
    ])
def _count_kernel(dstids, zrows, ones_rows, cnt_out, dst_v, obuf, acc2,
                  sem_i, sem_s):
  c = lax.axis_index("c")
  s = lax.axis_index("s")
  wid = c * NS + s
  arow = pl.multiple_of(s * ROWS_PER_TILE, ROWS_PER_TILE)
  pltpu.sync_copy(ones_rows, obuf)
  pltpu.sync_copy(zrows, acc2.at[pl.ds(arow, ROWS_PER_TILE)])
  plsc.subcore_barrier()
  ebase = wid * E_PER_TILE

  def idx_start(k, b):
    off = pl.multiple_of(ebase + k * CHUNK, CHUNK)
    pltpu.async_copy(dstids.at[pl.ds(off, CHUNK)], dst_v[b], sem_i[b])

  def idx_wait(b):
    pltpu.make_async_copy(dstids.at[pl.ds(0, CHUNK)], dst_v[b],
                          sem_i[b]).wait()

  def scatter_start(b):
    pltpu.async_copy(obuf, acc2.at[dst_v[b]], sem_s[b], add=True)

  def scatter_wait(b):
    pltpu.make_async_copy(obuf, acc2.at[dst_v[b]], sem_s[b]).wait()

  for b in range(CNBUF):
    idx_start(b, b)

  def group(g, carry):
    k0 = g * CNBUF
    for b in range(CNBUF):
      idx_wait(b)
      scatter_start(b)
    for b in range(CNBUF):
      scatter_wait(b)
      idx_start(k0 + CNBUF + b, b)
    return carry

  lax.fori_loop(0, CN_GROUPS - 1, group, 0)
  for b in range(CNBUF):
    idx_wait(b)
    scatter_start(b)
  for b in range(CNBUF):
    scatter_wait(b)
  plsc.subcore_barrier()
  pltpu.sync_copy(acc2.at[pl.ds(arow, ROWS_PER_TILE)], cnt_out.at[wid])


def _mm_body(x_ref, wl_ref, wr_ref, b_ref, t_ref, r_ref):
  xb = x_ref[...]
  t_ref[...] = jnp.dot(xb, wl_ref[...], preferred_element_type=jnp.float32)
  r_ref[...] = (jnp.dot(xb, wr_ref[...], preferred_element_type=jnp.float32)
                + b_ref[...])


def _rcp_cnt(c0_ref, c1_ref):
  cnt = c0_ref[...][:, 0:1] + c1_ref[...][:, 0:1]
  return 1.0 / jnp.maximum(cnt, 1.0)


def _combine_mm_body(p0_ref, p1_ref, c0_ref, c1_ref, r_ref, wl_ref, wr_ref,
                     b_ref, t_ref, rr_ref):
  mean = (p0_ref[...] + p1_ref[...]) * _rcp_cnt(c0_ref, c1_ref)
  h = jnp.maximum(mean + r_ref[...], 0.0)
  t_ref[...] = jnp.dot(h, wl_ref[...], preferred_element_type=jnp.float32)
  rr_ref[...] = (jnp.dot(h, wr_ref[...], preferred_element_type=jnp.float32)
                 + b_ref[...])


def _final_body(p0_ref, p1_ref, c0_ref, c1_ref, r_ref, o_ref):
  o_ref[...] = ((p0_ref[...] + p1_ref[...]) * _rcp_cnt(c0_ref, c1_ref)
                + r_ref[...])


_blk = lambda i: (i, 0)
_rep2 = lambda i: (0, 0)
_w_spec = pl.BlockSpec((D, D), _rep2)
_b_spec = pl.BlockSpec((1, D), _rep2)
_row_spec = pl.BlockSpec((NB, D), _blk)

_mm = pl.pallas_call(
    _mm_body,
    grid=(GRID,),
    in_specs=[_row_spec, _w_spec, _w_spec, _b_spec],
    out_specs=[_row_spec, _row_spec],
    out_shape=[jax.ShapeDtypeStruct((N_NODES, D), jnp.float32)] * 2,
)

_combine_mm = pl.pallas_call(
    _combine_mm_body,
    grid=(GRID,),
    in_specs=[_row_spec, _row_spec, _row_spec, _row_spec, _row_spec, _w_spec,
              _w_spec, _b_spec],
    out_specs=[_row_spec, _row_spec],
    out_shape=[jax.ShapeDtypeStruct((N_NODES, D), jnp.float32)] * 2,
)

_final = pl.pallas_call(
    _final_body,
    grid=(GRID,),
    in_specs=[_row_spec, _row_spec, _row_spec, _row_spec, _row_spec],
    out_specs=_row_spec,
    out_shape=jax.ShapeDtypeStruct((N_NODES, D), jnp.float32),
)


def kernel(x, edge_index, Wl1, Wr1, b1, Wl2, Wr2, b2, Wl3, Wr3, b3):
  src = edge_index[0].astype(jnp.int32)
  dst = edge_index[1].astype(jnp.int32)
  pad = E_PAD - N_EDGES
  src = jnp.concatenate([src, jnp.zeros((pad,), jnp.int32)])
  dst = jnp.concatenate([dst, jnp.full((pad,), N_NODES, jnp.int32)])
  zrows = jnp.zeros((ROWS_PER_TILE, D), jnp.float32)
  ones_rows = jnp.ones((CHUNK, D), jnp.float32)
  b1r, b2r, b3r = (b.reshape(1, D) for b in (b1, b2, b3))

  t1, r1 = _mm(x, Wl1, Wr1, b1r)
  cnt = _count_kernel(dst, zrows, ones_rows).reshape(NC, ACC_ROWS, D)
  c0, c1 = cnt[0], cnt[1]
  P1 = _seg_sum(t1, src, dst, zrows).reshape(NC, ACC_ROWS, D)
  t2, r2 = _combine_mm(P1[0], P1[1], c0, c1, r1, Wl2, Wr2, b2r)
  P2 = _seg_sum(t2, src, dst, zrows).reshape(NC, ACC_ROWS, D)
  t3, r3 = _combine_mm(P2[0], P2[1], c0, c1, r2, Wl3, Wr3, b3r)
  P3 = _seg_sum(t3, src, dst, zrows).reshape(NC, ACC_ROWS, D)
  return _final(P3[0], P3[1], c0, c1, r3)


# speedup vs baseline: 1.2141x; 1.2141x over previous
import functools

import jax
import jax.numpy as jnp
from jax import lax
from jax.experimental import pallas as pl
from jax.experimental.pallas import tpu as pltpu
from jax.experimental.pallas import tpu_sc as plsc

N_NODES = 10000
N_EDGES = 320000
D = 128

NC = 2
NS = 16
NW = NC * NS
E_PER_TILE = 10240
E_PAD = NW * E_PER_TILE
CHUNK = 128
N_CHUNKS = E_PER_TILE // CHUNK
ACC_ROWS = 10240
ROWS_PER_TILE = ACC_ROWS // NS

NB = 1024
GRID = (N_NODES + NB - 1) // NB

_mesh = lambda: plsc.VectorSubcoreMesh(core_axis_name="c", subcore_axis_name="s")


NBUF = 2
CNBUF = 4
CN_GROUPS = N_CHUNKS // CNBUF

A_CH = 120
B_CH = 2 * N_CHUNKS - A_CH


@functools.partial(
    pl.kernel,
    out_type=jax.ShapeDtypeStruct((NW, ROWS_PER_TILE, D), jnp.float32),
    mesh=_mesh(),
    scratch_types=[
        [pltpu.VMEM((CHUNK,), jnp.int32)] * NBUF,
        [pltpu.VMEM((CHUNK,), jnp.int32)] * NBUF,
        [pltpu.VMEM((CHUNK, D), jnp.float32)] * NBUF,
        pltpu.VMEM_SHARED((ACC_ROWS, D), jnp.float32),
        [pltpu.SemaphoreType.DMA] * NBUF,
        [pltpu.SemaphoreType.DMA] * NBUF,
        [pltpu.SemaphoreType.DMA] * NBUF,
    ])
def _seg_sum(table, src, dst, zrows, psum_out, src_v, dst_v, rows_v, acc,
             sem_i, sem_g, sem_s):
  c = lax.axis_index("c")
  s = lax.axis_index("s")
  wid = c * NS + s
  arow = pl.multiple_of(s * ROWS_PER_TILE, ROWS_PER_TILE)
  pltpu.sync_copy(zrows, acc.at[pl.ds(arow, ROWS_PER_TILE)])
  plsc.subcore_barrier()

  ebase = jnp.where(c == 0, s * (A_CH * CHUNK),
                    NS * (A_CH * CHUNK) + s * (B_CH * CHUNK))
  n_groups = jnp.where(c == 0, A_CH // NBUF, B_CH // NBUF)

  def idx_start(k, b):
    off = pl.multiple_of(ebase + k * CHUNK, CHUNK)
    pltpu.async_copy(src.at[pl.ds(off, CHUNK)], src_v[b], sem_i[b])
    pltpu.async_copy(dst.at[pl.ds(off, CHUNK)], dst_v[b], sem_i[b])

  def idx_wait(b):
    pltpu.make_async_copy(src.at[pl.ds(0, CHUNK)], src_v[b], sem_i[b]).wait()
    pltpu.make_async_copy(dst.at[pl.ds(0, CHUNK)], dst_v[b], sem_i[b]).wait()

  def gather_start(b):
    pltpu.async_copy(table.at[src_v[b]], rows_v[b], sem_g[b])

  def gather_wait(b):
    pltpu.make_async_copy(table.at[src_v[b]], rows_v[b], sem_g[b]).wait()

  def scatter_start(b):
    pltpu.async_copy(rows_v[b], acc.at[dst_v[b]], sem_s[b], add=True)

  def scatter_wait(b):
    pltpu.make_async_copy(rows_v[b], acc.at[dst_v[b]], sem_s[b]).wait()

  for b in range(NBUF):
    idx_start(b, b)
  for b in range(NBUF):
    idx_wait(b)
    gather_start(b)

  def group(g, carry):
    k0 = g * NBUF
    for b in range(NBUF):
      gather_wait(b)
      scatter_start(b)
    for b in range(NBUF):
      scatter_wait(b)
      idx_start(k0 + NBUF + b, b)
    for b in range(NBUF):
      idx_wait(b)
      gather_start(b)
    return carry

  lax.fori_loop(0, n_groups - 1, group, 0)
  for b in range(NBUF):
    gather_wait(b)
    scatter_start(b)
  for b in range(NBUF):
    scatter_wait(b)

  plsc.subcore_barrier()
  pltpu.sync_copy(acc.at[pl.ds(arow, ROWS_PER_TILE)], psum_out.at[wid])


@functools.partial(
    pl.kernel,
    out_type=jax.ShapeDtypeStruct((NW, ROWS_PER_TILE, D), jnp.float32),
    mesh=_mesh(),
    scratch_types=[
        [pltpu.VMEM((CHUNK,), jnp.int32)] * CNBUF,
        pltpu.VMEM((CHUNK, D), jnp.float32),
        pltpu.VMEM_SHARED((ACC_ROWS, D), jnp.float32),
        [pltpu.SemaphoreType.DMA] * CNBUF,
        [pltpu.SemaphoreType.DMA] * CNBUF,
    ])
def _count_kernel(dstids, zrows, ones_rows, cnt_out, dst_v, obuf, acc2,
                  sem_i, sem_s):
  c = lax.axis_index("c")
  s = lax.axis_index("s")
  wid = c * NS + s
  arow = pl.multiple_of(s * ROWS_PER_TILE, ROWS_PER_TILE)
  pltpu.sync_copy(ones_rows, obuf)
  pltpu.sync_copy(zrows, acc2.at[pl.ds(arow, ROWS_PER_TILE)])
  plsc.subcore_barrier()
  ebase = wid * E_PER_TILE

  def idx_start(k, b):
    off = pl.multiple_of(ebase + k * CHUNK, CHUNK)
    pltpu.async_copy(dstids.at[pl.ds(off, CHUNK)], dst_v[b], sem_i[b])

  def idx_wait(b):
    pltpu.make_async_copy(dstids.at[pl.ds(0, CHUNK)], dst_v[b],
                          sem_i[b]).wait()

  def scatter_start(b):
    pltpu.async_copy(obuf, acc2.at[dst_v[b]], sem_s[b], add=True)

  def scatter_wait(b):
    pltpu.make_async_copy(obuf, acc2.at[dst_v[b]], sem_s[b]).wait()

  for b in range(CNBUF):
    idx_start(b, b)

  def group(g, carry):
    k0 = g * CNBUF
    for b in range(CNBUF):
      idx_wait(b)
      scatter_start(b)
    for b in range(CNBUF):
      scatter_wait(b)
      idx_start(k0 + CNBUF + b, b)
    return carry

  lax.fori_loop(0, CN_GROUPS - 1, group, 0)
  for b in range(CNBUF):
    idx_wait(b)
    scatter_start(b)
  for b in range(CNBUF):
    scatter_wait(b)
  plsc.subcore_barrier()
  pltpu.sync_copy(acc2.at[pl.ds(arow, ROWS_PER_TILE)], cnt_out.at[wid])


def _mm_body(x_ref, wl_ref, wr_ref, b_ref, t_ref, r_ref):
  xb = x_ref[...]
  t_ref[...] = jnp.dot(xb, wl_ref[...], preferred_element_type=jnp.float32)
  r_ref[...] = (jnp.dot(xb, wr_ref[...], preferred_element_type=jnp.float32)
                + b_ref[...])


def _rcp_cnt(c0_ref, c1_ref):
  cnt = c0_ref[...][:, 0:1] + c1_ref[...][:, 0:1]
  return 1.0 / jnp.maximum(cnt, 1.0)


def _combine_mm_body(p0_ref, p1_ref, c0_ref, c1_ref, r_ref, wl_ref, wr_ref,
                     b_ref, t_ref, rr_ref):
  mean = (p0_ref[...] + p1_ref[...]) * _rcp_cnt(c0_ref, c1_ref)
  h = jnp.maximum(mean + r_ref[...], 0.0)
  t_ref[...] = jnp.dot(h, wl_ref[...], preferred_element_type=jnp.float32)
  rr_ref[...] = (jnp.dot(h, wr_ref[...], preferred_element_type=jnp.float32)
                 + b_ref[...])


def _final_body(p0_ref, p1_ref, c0_ref, c1_ref, r_ref, o_ref):
  o_ref[...] = ((p0_ref[...] + p1_ref[...]) * _rcp_cnt(c0_ref, c1_ref)
                + r_ref[...])


_blk = lambda i: (i, 0)
_rep2 = lambda i: (0, 0)
_w_spec = pl.BlockSpec((D, D), _rep2)
_b_spec = pl.BlockSpec((1, D), _rep2)
_row_spec = pl.BlockSpec((NB, D), _blk)

_mm = pl.pallas_call(
    _mm_body,
    grid=(GRID,),
    in_specs=[_row_spec, _w_spec, _w_spec, _b_spec],
    out_specs=[_row_spec, _row_spec],
    out_shape=[jax.ShapeDtypeStruct((N_NODES, D), jnp.float32)] * 2,
)

_combine_mm = pl.pallas_call(
    _combine_mm_body,
    grid=(GRID,),
    in_specs=[_row_spec, _row_spec, _row_spec, _row_spec, _row_spec, _w_spec,
              _w_spec, _b_spec],
    out_specs=[_row_spec, _row_spec],
    out_shape=[jax.ShapeDtypeStruct((N_NODES, D), jnp.float32)] * 2,
)

_final = pl.pallas_call(
    _final_body,
    grid=(GRID,),
    in_specs=[_row_spec, _row_spec, _row_spec, _row_spec, _row_spec],
    out_specs=_row_spec,
    out_shape=jax.ShapeDtypeStruct((N_NODES, D), jnp.float32),
)


def kernel(x, edge_index, Wl1, Wr1, b1, Wl2, Wr2, b2, Wl3, Wr3, b3):
  src = edge_index[0].astype(jnp.int32)
  dst = edge_index[1].astype(jnp.int32)
  pad = E_PAD - N_EDGES
  src = jnp.concatenate([src, jnp.zeros((pad,), jnp.int32)])
  dst = jnp.concatenate([dst, jnp.full((pad,), N_NODES, jnp.int32)])
  zrows = jnp.zeros((ROWS_PER_TILE, D), jnp.float32)
  ones_rows = jnp.ones((CHUNK, D), jnp.float32)
  b1r, b2r, b3r = (b.reshape(1, D) for b in (b1, b2, b3))

  t1, r1 = _mm(x, Wl1, Wr1, b1r)
  cnt = _count_kernel(dst, zrows, ones_rows).reshape(NC, ACC_ROWS, D)
  c0, c1 = cnt[0], cnt[1]
  P1 = _seg_sum(t1, src, dst, zrows).reshape(NC, ACC_ROWS, D)
  t2, r2 = _combine_mm(P1[0], P1[1], c0, c1, r1, Wl2, Wr2, b2r)
  P2 = _seg_sum(t2, src, dst, zrows).reshape(NC, ACC_ROWS, D)
  t3, r3 = _combine_mm(P2[0], P2[1], c0, c1, r2, Wl3, Wr3, b3r)
  P3 = _seg_sum(t3, src, dst, zrows).reshape(NC, ACC_ROWS, D)
  return _final(P3[0], P3[1], c0, c1, r3)
